# baseline (speedup 1.0000x reference)
import jax
import jax.numpy as jnp
from jax.experimental import pallas as pl

_BB = 4
_HP = 512


def _add_kernel(pa_ref, pb_ref, t_ref, o_ref):
    o_ref[:, :_HP] = pa_ref[...] + t_ref[:_HP]
    o_ref[:, _HP:] = pb_ref[...] + t_ref[_HP:]


def kernel(patches, table):
    B, P, E = patches.shape
    return pl.pallas_call(
        _add_kernel,
        grid=(B // _BB,),
        in_specs=[
            pl.BlockSpec((_BB, _HP, E), lambda b: (b, 0, 0)),
            pl.BlockSpec((_BB, _HP, E), lambda b: (b, 1, 0)),
            pl.BlockSpec((P, E), lambda b: (0, 0)),
        ],
        out_specs=pl.BlockSpec((_BB, P, E), lambda b: (b, 0, 0)),
        out_shape=jax.ShapeDtypeStruct((B, P, E), patches.dtype),
    )(patches, patches, table)
